# Initial kernel scaffold; baseline (speedup 1.0000x reference)
import jax
import jax.numpy as jnp
from jax.experimental import pallas as pl


def kernel(x, w, b):
    raise NotImplementedError("write your pallas kernel here")


# speedup vs baseline: 4.4604x; 4.4604x over previous
import jax
import jax.numpy as jnp
from jax.experimental import pallas as pl
from jax.experimental.pallas import tpu as pltpu

_EPS = 1e-08
_N, _C, _H, _W = 8, 64, 32, 32
_F = 64
_P = _H * _W
_NBINS = 9
_TOTAL = 9 * _N * _F * (_C // 8) * _P


def _roll_lanes(v, shift):
    s = shift % _P
    if s == 0:
        return v
    return jnp.concatenate([v[:, s:], v[:, :s]], axis=1)


def _fused_kernel(x_ref, wt_ref, thr_ref, out_ref, hist_ref):
    xa = x_ref[0]

    p_idx = jax.lax.broadcasted_iota(jnp.int32, (1, _P), 1)
    col = p_idx & (_W - 1)
    row_lo = p_idx >= _W
    row_hi = p_idx < (_P - _W)
    col_lo = col != 0
    col_hi = col != (_W - 1)

    out_acc = jnp.zeros((_F, _P), jnp.float32)
    bins = [jnp.zeros((8, _P), jnp.int32) for _ in range(8)]

    for t in range(9):
        di, dj = t // 3 - 1, t % 3 - 1
        off = di * _W + dj
        xs = _roll_lanes(xa, off)
        vm = None
        for cond in ((row_lo if di < 0 else (row_hi if di > 0 else None)),
                     (col_lo if dj < 0 else (col_hi if dj > 0 else None))):
            if cond is not None:
                vm = cond if vm is None else (vm & cond)
        xsm = xs if vm is None else jnp.where(vm, xs, 0.0)

        out_acc = out_acc + jnp.dot(wt_ref[t], xsm,
                                    preferred_element_type=jnp.float32)

        xab = jnp.abs(xsm).reshape(8, 8, _P)
        for fc in range(8):
            tb = thr_ref[t, fc * 8:(fc + 1) * 8]
            m = xab[None] > tb[..., None]
            cnt = jnp.where(m, 1, 0).sum(axis=1)
            for k in range(8):
                bins[k] = bins[k] + jnp.where(cnt == k, 1, 0).sum(axis=0)

    out_ref[0] = out_acc
    for k in range(8):
        hist_ref[0, 0, k] = jnp.sum(bins[k])


def kernel(x, w, b):
    del b
    xp = x.reshape(_N, 8, 8, _P).transpose(0, 2, 1, 3).reshape(_N, _C, _P)
    wq = w.transpose(2, 3, 0, 1).reshape(9, _F, 8, 8).transpose(0, 1, 3, 2)
    wt = wq.reshape(9, _F, _C)
    thr = _EPS / jnp.abs(wq)

    out, hist_n = pl.pallas_call(
        _fused_kernel,
        grid=(_N,),
        in_specs=[
            pl.BlockSpec((1, _C, _P), lambda n: (n, 0, 0)),
            pl.BlockSpec((9, _F, _C), lambda n: (0, 0, 0)),
            pl.BlockSpec((9, _F, 8, 8), lambda n: (0, 0, 0, 0)),
        ],
        out_specs=[
            pl.BlockSpec((1, _F, _P), lambda n: (n, 0, 0)),
            pl.BlockSpec((1, 1, 8), lambda n: (n, 0, 0),
                         memory_space=pltpu.SMEM),
        ],
        out_shape=[
            jax.ShapeDtypeStruct((_N, _F, _P), jnp.float32),
            jax.ShapeDtypeStruct((_N, 1, 8), jnp.int32),
        ],
        compiler_params=pltpu.CompilerParams(
            dimension_semantics=("parallel",),
        ),
        name="myconv_hist",
    )(xp, wt, thr)

    lo = hist_n[:, 0, :].sum(axis=0)
    hist = jnp.concatenate([lo, (_TOTAL - lo.sum())[None]])
    return out.reshape(_N, _F, _H, _W), hist.astype(jnp.int64)


# speedup vs baseline: 7.3795x; 1.6544x over previous
import jax
import jax.numpy as jnp
from jax.experimental import pallas as pl
from jax.experimental.pallas import tpu as pltpu

_EPS = 1e-08
_N, _C, _H, _W = 8, 64, 32, 32
_F = 64
_P = _H * _W
_NBINS = 9
_TOTAL = 9 * _N * _F * (_C // 8) * _P


def _roll_lanes(v, shift):
    s = shift % _P
    if s == 0:
        return v
    return jnp.concatenate([v[:, s:], v[:, :s]], axis=1)


def _fused_kernel(x_ref, wt_ref, thr_ref, out_ref, hist_ref):
    xa = x_ref[0]

    p_idx = jax.lax.broadcasted_iota(jnp.int32, (1, _P), 1)
    col = p_idx & (_W - 1)
    row_lo = p_idx >= _W
    row_hi = p_idx < (_P - _W)
    col_lo = col != 0
    col_hi = col != (_W - 1)

    out_acc = jnp.zeros((_F, _P), jnp.float32)
    bins = [jnp.zeros((8, _P), jnp.int32) for _ in range(8)]
    acc_even = jnp.zeros((8, _P), jnp.int32)
    acc_odd = jnp.zeros((8, _P), jnp.int32)

    for t in range(9):
        di, dj = t // 3 - 1, t % 3 - 1
        off = di * _W + dj
        xs = _roll_lanes(xa, off)
        vm = None
        for cond in ((row_lo if di < 0 else (row_hi if di > 0 else None)),
                     (col_lo if dj < 0 else (col_hi if dj > 0 else None))):
            if cond is not None:
                vm = cond if vm is None else (vm & cond)
        xsm = xs if vm is None else jnp.where(vm, xs, 0.0)

        out_acc = out_acc + jnp.dot(wt_ref[t], xsm,
                                    preferred_element_type=jnp.float32)

        xab = jnp.abs(xsm).reshape(8, 8, _P)
        for fc in range(8):
            tb = thr_ref[t, fc * 8:(fc + 1) * 8]
            m = xab[None] > tb[..., None]
            cnt = jnp.where(m, 1, 0).sum(axis=1)
            pw = jnp.int32(1) << (cnt << 2)
            s = jnp.where(cnt < 8, pw, 0).sum(axis=0)
            acc_even = acc_even + (s & 0x0F0F0F0F)
            acc_odd = acc_odd + (jax.lax.shift_right_logical(s, 4) & 0x0F0F0F0F)
        if t % 3 == 2:
            for i in range(4):
                bins[2 * i] = bins[2 * i] + \
                    (jax.lax.shift_right_logical(acc_even, 8 * i) & 255)
                bins[2 * i + 1] = bins[2 * i + 1] + \
                    (jax.lax.shift_right_logical(acc_odd, 8 * i) & 255)
            acc_even = jnp.zeros((8, _P), jnp.int32)
            acc_odd = jnp.zeros((8, _P), jnp.int32)

    out_ref[0] = out_acc
    for k in range(8):
        hist_ref[0, 0, k] = jnp.sum(bins[k])


def kernel(x, w, b):
    del b
    xp = x.reshape(_N, 8, 8, _P).transpose(0, 2, 1, 3).reshape(_N, _C, _P)
    wq = w.transpose(2, 3, 0, 1).reshape(9, _F, 8, 8).transpose(0, 1, 3, 2)
    wt = wq.reshape(9, _F, _C)
    thr = _EPS / jnp.abs(wq)

    out, hist_n = pl.pallas_call(
        _fused_kernel,
        grid=(_N,),
        in_specs=[
            pl.BlockSpec((1, _C, _P), lambda n: (n, 0, 0)),
            pl.BlockSpec((9, _F, _C), lambda n: (0, 0, 0)),
            pl.BlockSpec((9, _F, 8, 8), lambda n: (0, 0, 0, 0)),
        ],
        out_specs=[
            pl.BlockSpec((1, _F, _P), lambda n: (n, 0, 0)),
            pl.BlockSpec((1, 1, 8), lambda n: (n, 0, 0),
                         memory_space=pltpu.SMEM),
        ],
        out_shape=[
            jax.ShapeDtypeStruct((_N, _F, _P), jnp.float32),
            jax.ShapeDtypeStruct((_N, 1, 8), jnp.int32),
        ],
        compiler_params=pltpu.CompilerParams(
            dimension_semantics=("parallel",),
        ),
        name="myconv_hist",
    )(xp, wt, thr)

    lo = hist_n[:, 0, :].sum(axis=0)
    hist = jnp.concatenate([lo, (_TOTAL - lo.sum())[None]])
    return out.reshape(_N, _F, _H, _W), hist.astype(jnp.int64)
